# baseline (speedup 1.0000x reference)
import functools

import jax
import jax.numpy as jnp
from jax import lax
from jax.experimental import pallas as pl

B, NB, PL, H, F = 16, 64, 24, 1024, 4096
M = B * NB
BM = 128
NSTEP = M // BM
BF = 2 * (F // NSTEP)


def _fused_body(vis_ref, lang_ref, invlen_ref, wv_ref, bv_ref, wl_ref, bl_ref,
                lmap_ref, vmap_ref):
    i = pl.program_id(0)

    pooled = jnp.sum(lang_ref[...], axis=1) * invlen_ref[...]
    lmap_ref[...] = (
        lax.dot_general(pooled, wl_ref[...], (((1,), (1,)), ((), ())),
                        preferred_element_type=jnp.float32)
        + bl_ref[...]
    )

    half = lax.rem(i, 2) * (BF // 2)
    prod = lax.dot_general(vis_ref[:, pl.ds(half, BF // 2)],
                           wv_ref[:, pl.ds(half, BF // 2)],
                           (((1,), (1,)), ((), ())),
                           preferred_element_type=jnp.float32)

    @pl.when(i == 0)
    def _():
        vmap_ref[...] = prod + bv_ref[...]

    @pl.when(i > 0)
    def _():
        vmap_ref[...] += prod


@functools.partial(jax.jit, static_argnames=())
def kernel(vision, language, phrase_lengths, Wv, bv, Wl, bl):
    vis = vision.reshape(M, F)
    lang = language.reshape(M, PL, H)
    inv_len = (1.0 / phrase_lengths.astype(jnp.float32)).reshape(M, 1)

    lmap, vmap = pl.pallas_call(
        _fused_body,
        grid=(NSTEP,),
        in_specs=[
            pl.BlockSpec((M, BF), lambda i: (0, i // 2)),
            pl.BlockSpec((BM, PL, H), lambda i: (i, 0, 0)),
            pl.BlockSpec((BM, 1), lambda i: (i, 0)),
            pl.BlockSpec((H, BF), lambda i: (0, i // 2)),
            pl.BlockSpec((1, H), lambda i: (0, 0)),
            pl.BlockSpec((H, H), lambda i: (0, 0)),
            pl.BlockSpec((1, H), lambda i: (0, 0)),
        ],
        out_specs=[
            pl.BlockSpec((BM, H), lambda i: (i, 0)),
            pl.BlockSpec((M, H), lambda i: (0, 0)),
        ],
        out_shape=[
            jax.ShapeDtypeStruct((M, H), jnp.float32),
            jax.ShapeDtypeStruct((M, H), jnp.float32),
        ],
    )(vis, lang, inv_len, Wv, bv.reshape(1, H), Wl, bl.reshape(1, H))

    return (lmap.reshape(B, NB, H), vmap.reshape(B, NB, H))


# speedup vs baseline: 1.0805x; 1.0805x over previous
import functools

import jax
import jax.numpy as jnp
from jax import lax
from jax.experimental import pallas as pl

B, NB, PL, H, F = 16, 64, 24, 1024, 4096
M = B * NB
BM = 128
NSTEP = M // BM
BF = F // NSTEP


def _fused_body(vis_ref, lang_ref, invlen_ref, wv_ref, bv_ref, wl_ref, bl_ref,
                lmap_ref, vmap_ref):
    i = pl.program_id(0)

    pooled = jnp.sum(lang_ref[...], axis=1) * invlen_ref[...]
    lmap_ref[...] = (
        lax.dot_general(pooled, wl_ref[...], (((1,), (1,)), ((), ())),
                        preferred_element_type=jnp.float32)
        + bl_ref[...]
    )

    prod = lax.dot_general(vis_ref[...], wv_ref[...], (((1,), (1,)), ((), ())),
                           preferred_element_type=jnp.float32)

    @pl.when(i == 0)
    def _():
        vmap_ref[...] = prod + bv_ref[...]

    @pl.when(i > 0)
    def _():
        vmap_ref[...] += prod


@functools.partial(jax.jit, static_argnames=())
def kernel(vision, language, phrase_lengths, Wv, bv, Wl, bl):
    vis = vision.reshape(M, F)
    lang = language.reshape(M, PL, H)
    inv_len = (1.0 / phrase_lengths.astype(jnp.float32)).reshape(M, 1)

    lmap, vmap = pl.pallas_call(
        _fused_body,
        grid=(NSTEP,),
        in_specs=[
            pl.BlockSpec((M, BF), lambda i: (0, i)),
            pl.BlockSpec((BM, PL, H), lambda i: (i, 0, 0)),
            pl.BlockSpec((BM, 1), lambda i: (i, 0)),
            pl.BlockSpec((H, BF), lambda i: (0, i)),
            pl.BlockSpec((1, H), lambda i: (0, 0)),
            pl.BlockSpec((H, H), lambda i: (0, 0)),
            pl.BlockSpec((1, H), lambda i: (0, 0)),
        ],
        out_specs=[
            pl.BlockSpec((BM, H), lambda i: (i, 0)),
            pl.BlockSpec((M, H), lambda i: (0, 0)),
        ],
        out_shape=[
            jax.ShapeDtypeStruct((M, H), jnp.float32),
            jax.ShapeDtypeStruct((M, H), jnp.float32),
        ],
    )(vis, lang, inv_len, Wv, bv.reshape(1, H), Wl, bl.reshape(1, H))

    return (lmap.reshape(B, NB, H), vmap.reshape(B, NB, H))


# speedup vs baseline: 1.1236x; 1.0399x over previous
import functools

import jax
import jax.numpy as jnp
from jax import lax
from jax.experimental import pallas as pl
from jax.experimental.pallas import tpu as pltpu

B, NB, PL, H, F = 16, 64, 24, 1024, 4096
M = B * NB
BM = 128
NSTEP = M // BM
BF = F // NSTEP


def _fused_body(vis_ref, lang_ref, invlen_ref, wv_ref, wl_hbm, bv_hbm, bl_hbm,
                lmap_ref, vmap_ref, wl_v, bv_v, bl_v, sem_wl, sem_bv, sem_bl):
    i = pl.program_id(0)

    @pl.when(i == 0)
    def _():
        pltpu.async_copy(wl_hbm, wl_v, sem_wl)
        pltpu.async_copy(bv_hbm, bv_v, sem_bv)
        pltpu.async_copy(bl_hbm, bl_v, sem_bl)

    pooled = jnp.sum(lang_ref[...], axis=1) * invlen_ref[...]

    prod = lax.dot_general(vis_ref[...], wv_ref[...], (((1,), (1,)), ((), ())),
                           preferred_element_type=jnp.float32)

    @pl.when(i == 0)
    def _():
        pltpu.make_async_copy(wl_hbm, wl_v, sem_wl).wait()
        pltpu.make_async_copy(bv_hbm, bv_v, sem_bv).wait()
        pltpu.make_async_copy(bl_hbm, bl_v, sem_bl).wait()
        vmap_ref[...] = prod + bv_v[...]

    @pl.when(i > 0)
    def _():
        vmap_ref[...] += prod

    lmap_ref[...] = (
        lax.dot_general(pooled, wl_v[...], (((1,), (1,)), ((), ())),
                        preferred_element_type=jnp.float32)
        + bl_v[...]
    )


@functools.partial(jax.jit, static_argnames=())
def kernel(vision, language, phrase_lengths, Wv, bv, Wl, bl):
    vis = vision.reshape(M, F)
    lang = language.reshape(M, PL, H)
    inv_len = (1.0 / phrase_lengths.astype(jnp.float32)).reshape(M, 1)

    lmap, vmap = pl.pallas_call(
        _fused_body,
        grid=(NSTEP,),
        in_specs=[
            pl.BlockSpec((M, BF), lambda i: (0, i)),
            pl.BlockSpec((BM, PL, H), lambda i: (i, 0, 0)),
            pl.BlockSpec((BM, 1), lambda i: (i, 0)),
            pl.BlockSpec((H, BF), lambda i: (0, i)),
            pl.BlockSpec(memory_space=pl.ANY),
            pl.BlockSpec(memory_space=pl.ANY),
            pl.BlockSpec(memory_space=pl.ANY),
        ],
        out_specs=[
            pl.BlockSpec((BM, H), lambda i: (i, 0)),
            pl.BlockSpec((M, H), lambda i: (0, 0)),
        ],
        out_shape=[
            jax.ShapeDtypeStruct((M, H), jnp.float32),
            jax.ShapeDtypeStruct((M, H), jnp.float32),
        ],
        scratch_shapes=[
            pltpu.VMEM((H, H), jnp.float32),
            pltpu.VMEM((1, H), jnp.float32),
            pltpu.VMEM((1, H), jnp.float32),
            pltpu.SemaphoreType.DMA,
            pltpu.SemaphoreType.DMA,
            pltpu.SemaphoreType.DMA,
        ],
    )(vis, lang, inv_len, Wv, Wl, bv.reshape(1, H), bl.reshape(1, H))

    return (lmap.reshape(B, NB, H), vmap.reshape(B, NB, H))
